# Initial kernel scaffold; baseline (speedup 1.0000x reference)
import jax
import jax.numpy as jnp
from jax.experimental import pallas as pl


def kernel(item_id, item_name_tokens, item_gics, emb_id, emb_name, emb_gics):
    raise NotImplementedError("write your pallas kernel here")


# speedup vs baseline: 12.1827x; 12.1827x over previous
import jax
import jax.numpy as jnp
from jax import lax
from jax.experimental import pallas as pl
from jax.experimental.pallas import tpu as pltpu
from jax.experimental.pallas import tpu_sc as plsc

B = 16384
L = 20
NC, NS = 2, 16
NW = NC * NS
PB = B // NW
IDXW = 80
IPW = PB * L // IDXW
G = 8
CI = PB // G
IPG = IPW // G
TPG = CI * L


def _body(idf, tokf, gicsf, emb_id, emb_name, emb_gics, out,
          tok_v, idid_v, idg_v, idrow_v, grow_v,
          rows0, rows1, out0, out1, row0_v, rec_v, nz_v,
          sem_idg, sem_n0, sem_n1, sem_out):
    wid = lax.axis_index("s") * NC + lax.axis_index("c")
    base = wid * PB
    rows = (rows0, rows1)
    outb = (out0, out1)
    sem_n = (sem_n0, sem_n1)

    pltpu.sync_copy(tokf.at[pl.ds(base * L, PB * L)], tok_v)
    pltpu.sync_copy(idf.at[pl.ds(base, PB)], idid_v)
    pltpu.sync_copy(gicsf.at[pl.ds(base, PB)], idg_v)
    pltpu.sync_copy(emb_name.at[pl.ds(0, 1)], row0_v)

    idg_copies = []
    for t in range(4):
        idg_copies.append(pltpu.async_copy(
            emb_id.at[idid_v.at[pl.ds(t * 128, 128)]],
            idrow_v.at[pl.ds(t * 128, 128)], sem_idg))
        idg_copies.append(pltpu.async_copy(
            emb_gics.at[idg_v.at[pl.ds(t * 128, 128)]],
            grow_v.at[pl.ds(t * 128, 128)], sem_idg))

    def fire_group(g):
        buf = g % 2
        return [pltpu.async_copy(
            emb_name.at[tok_v.at[pl.ds((g * IPG + t) * IDXW, IDXW)]],
            rows[buf].at[pl.ds(t * IDXW, IDXW)], sem_n[buf])
            for t in range(IPG)]

    pend = fire_group(0)
    lane = lax.iota(jnp.int32, 16)
    lane_lo = lane < 8
    col8 = lane & 7
    sidx0 = lane + jnp.where(lane_lo, 0, 16)
    row0 = row0_v[0, :]

    def cnt_body(k, carry):
        lidx = (k * 16 + lane) * L
        cnt = jnp.zeros((16,), jnp.float32)
        for j in range(L):
            tokj = plsc.load_gather(tok_v, [lidx + j])
            cnt = cnt + jnp.where(tokj != 0, 1.0, 0.0).astype(jnp.float32)
        rec_v[pl.ds(k * 16, 16)] = 1.0 / jnp.maximum(cnt, jnp.float32(1.0))
        nz_v[pl.ds(k * 16, 16)] = jnp.float32(L) - cnt
        return carry

    lax.fori_loop(0, PB // 16, cnt_body, 0)
    for cp in idg_copies:
        cp.wait()
    out_copies = []
    for g in range(G):
        buf = g % 2
        nxt = fire_group(g + 1) if g + 1 < G else None
        for cp in pend:
            cp.wait()
        pend = nxt
        if g >= 2:
            out_copies[g - 2].wait()

        def item_body(i, carry, g=g, buf=buf):
            t0 = i * L
            acc = rows[buf][t0, :]
            for j in range(1, L):
                acc = acc + rows[buf][t0 + j, :]
            gi = g * CI + i
            girow = jnp.full((16,), gi, dtype=jnp.int32)
            nz = plsc.load_gather(nz_v, [girow])
            rec = plsc.load_gather(rec_v, [girow])
            name = (acc - nz * row0) * rec
            idv = plsc.load_gather(idrow_v, [girow, col8])
            gicsv = plsc.load_gather(grow_v, [girow, col8])
            merged = jnp.where(lane_lo, idv, gicsv)
            plsc.store_scatter(outb[buf], [i * 32 + sidx0], merged)
            outb[buf][pl.ds(i * 32 + 8, 16)] = name
            return carry

        lax.fori_loop(0, CI, item_body, 0)
        out_copies.append(pltpu.async_copy(
            outb[buf], out.at[pl.ds((base + g * CI) * 32, CI * 32)],
            sem_out))

    for g in range(G - 2, G):
        out_copies[g].wait()


@jax.jit
def _run(idf, tokf, gicsf, emb_id, emb_name, emb_gics):
    mesh = plsc.VectorSubcoreMesh(
        core_axis_name="c", subcore_axis_name="s",
        num_cores=NC, num_subcores=NS)
    return pl.kernel(
        _body,
        out_type=jax.ShapeDtypeStruct((B * 32,), jnp.float32),
        mesh=mesh,
        compiler_params=pltpu.CompilerParams(
            needs_layout_passes=False, use_tc_tiling_on_sc=False),
        scratch_types=[
            pltpu.VMEM((PB * L,), jnp.int32),
            pltpu.VMEM((PB,), jnp.int32),
            pltpu.VMEM((PB,), jnp.int32),
            pltpu.VMEM((PB, 8), jnp.float32),
            pltpu.VMEM((PB, 8), jnp.float32),
            pltpu.VMEM((TPG, 16), jnp.float32),
            pltpu.VMEM((TPG, 16), jnp.float32),
            pltpu.VMEM((CI * 32,), jnp.float32),
            pltpu.VMEM((CI * 32,), jnp.float32),
            pltpu.VMEM((1, 16), jnp.float32),
            pltpu.VMEM((PB,), jnp.float32),
            pltpu.VMEM((PB,), jnp.float32),
            pltpu.SemaphoreType.DMA,
            pltpu.SemaphoreType.DMA,
            pltpu.SemaphoreType.DMA,
            pltpu.SemaphoreType.DMA,
        ],
    )(idf, tokf, gicsf, emb_id, emb_name, emb_gics)


def kernel(item_id, item_name_tokens, item_gics, emb_id, emb_name, emb_gics):
    idf = item_id.astype(jnp.int32).reshape(B)
    tokf = item_name_tokens.astype(jnp.int32).reshape(B * L)
    gicsf = item_gics.astype(jnp.int32).reshape(B)
    out = _run(idf, tokf, gicsf, emb_id, emb_name, emb_gics)
    return out.reshape(B, 32)


# speedup vs baseline: 19.1423x; 1.5713x over previous
import jax
import jax.numpy as jnp
from jax import lax
from jax.experimental import pallas as pl
from jax.experimental.pallas import tpu as pltpu
from jax.experimental.pallas import tpu_sc as plsc

B = 16384
L = 20
VID = 100001
VIDP = 100008
VT = 10000
VG = 1001
VGP = 1008
NC, NS = 2, 16
NW = NC * NS
PB = B // NW

IDC = 3200
IDALIGN = 99968
IDTAIL = VID - IDALIGN
NMC = 384
NM_FULL = VT // NMC
NMALIGN = NM_FULL * NMC
NMTAIL = VT - NMALIGN

G = 8
CI = PB // G

_MESH = dict(core_axis_name="c", subcore_axis_name="s",
             num_cores=NC, num_subcores=NS)


def _detile_body(tokT, idT, nameT, idtail, nametail,
                 tokf, idf, namef,
                 tok_v, tokout_v, id_v, idout_v, nmin_v, nmout_v,
                 tail_v, ntail_v, sem_in, sem_out):
    wid = lax.axis_index("s") * NC + lax.axis_index("c")
    lane = lax.iota(jnp.int32, 16)

    pltpu.sync_copy(tokT.at[:, pl.ds(wid * PB, PB)], tok_v)

    def tok_body(c, carry):
        for j in range(L):
            tokout_v[pl.ds(j * PB + c * 16, 16)] = tok_v[j, pl.ds(c * 16, 16)]
        return carry

    lax.fori_loop(0, PB // 16, tok_body, 0)
    tok_cp = [pltpu.async_copy(tokout_v.at[pl.ds(j * PB, PB)],
                               tokf.at[pl.ds(j * B + wid * PB, PB)], sem_out)
              for j in range(L)]

    def _id_chunk(off, ncols):
        pltpu.sync_copy(idT.at[:, pl.ds(off, ncols)],
                        id_v.at[:, pl.ds(0, ncols)])

        def id_body(c, carry):
            for d in range(8):
                idout_v[pl.ds(d * IDC + c * 16, 16)] = id_v[d, pl.ds(c * 16, 16)]
            return carry

        lax.fori_loop(0, ncols // 16, id_body, 0)
        cps = [pltpu.async_copy(
            idout_v.at[pl.ds(d * IDC, ncols)],
            idf.at[pl.ds(d * VIDP + off, ncols)], sem_out)
            for d in range(8)]
        for cp in cps:
            cp.wait()

    @pl.when(wid < 31)
    def _id_full():
        _id_chunk(wid * IDC, IDC)

    @pl.when(wid == 31)
    def _id_last():
        _id_chunk(31 * IDC, IDALIGN - 31 * IDC)
        pltpu.sync_copy(idtail, tail_v)
        cps = [pltpu.async_copy(
            tail_v.at[pl.ds(d * 40, 40)],
            idf.at[pl.ds(d * VIDP + IDALIGN, 40)], sem_out)
            for d in range(8)]
        for cp in cps:
            cp.wait()

    @pl.when(wid < NM_FULL)
    def _name_full():
        off_cols = wid * NMC
        pltpu.sync_copy(nameT.at[:, pl.ds(off_cols, NMC)], nmin_v)

        def row_body(t, carry):
            v = plsc.load_gather(nmin_v, [lane, jnp.full((16,), 0, jnp.int32) + t])
            nmout_v[pl.ds(t * 16, 16)] = v
            return carry

        lax.fori_loop(0, NMC, row_body, 0)
        pltpu.sync_copy(nmout_v, namef.at[pl.ds(off_cols * 16, NMC * 16)])

    @pl.when(wid == NM_FULL)
    def _name_last():
        pltpu.sync_copy(nametail, ntail_v)
        pltpu.sync_copy(ntail_v, namef.at[pl.ds(NMALIGN * 16, NMTAIL * 16)])

    for cp in tok_cp:
        cp.wait()


@jax.jit
def _detile(tokT, idT, nameT, idtail, nametail):
    mesh = plsc.VectorSubcoreMesh(**_MESH)
    return pl.kernel(
        _detile_body,
        out_type=(
            jax.ShapeDtypeStruct((B * L,), jnp.int32),
            jax.ShapeDtypeStruct((8 * VIDP,), jnp.float32),
            jax.ShapeDtypeStruct((VT * 16,), jnp.float32),
        ),
        mesh=mesh,
        scratch_types=[
            pltpu.VMEM((L, PB), jnp.int32),
            pltpu.VMEM((L * PB,), jnp.int32),
            pltpu.VMEM((8, IDC), jnp.float32),
            pltpu.VMEM((8 * IDC,), jnp.float32),
            pltpu.VMEM((16, NMC), jnp.float32),
            pltpu.VMEM((NMC * 16,), jnp.float32),
            pltpu.VMEM((320,), jnp.float32),
            pltpu.VMEM((NMTAIL * 16,), jnp.float32),
            pltpu.SemaphoreType.DMA,
            pltpu.SemaphoreType.DMA,
        ],
        compiler_params=pltpu.CompilerParams(
            needs_layout_passes=False, use_tc_tiling_on_sc=True),
    )(tokT, idT, nameT, idtail, nametail)


def _gather_body(idsf, tok2d, gicsids, idf, name2d, gicsf, out,
                 tok_v, idid_v, gid_v, gicstab_v, idxid_v, idrow_v,
                 rows0, rows1, out_v, row0_v, rec_v, nz_v,
                 sem_id, sem_n0, sem_n1, sem_out):
    wid = lax.axis_index("s") * NC + lax.axis_index("c")
    base = wid * PB
    rows = (rows0, rows1)
    sem_n = (sem_n0, sem_n1)
    lane = lax.iota(jnp.int32, 16)
    lane_lo = lane < 8
    col8 = lane & 7

    pltpu.sync_copy(tok2d.at[:, pl.ds(base, PB)], tok_v)
    pltpu.sync_copy(idsf.at[pl.ds(base, PB)], idid_v)
    pltpu.sync_copy(gicsids.at[pl.ds(base, PB)], gid_v)
    pltpu.sync_copy(gicsf, gicstab_v)
    pltpu.sync_copy(name2d.at[pl.ds(0, 1)], row0_v)

    def idx_body(k, carry):
        i0 = k * 16
        idv = idid_v[pl.ds(i0, 16)]
        pos = (i0 + lane) * 8
        for d in range(8):
            plsc.store_scatter(idxid_v, [pos + d], idv + d * VIDP)
        return carry

    lax.fori_loop(0, PB // 16, idx_body, 0)

    id_cp = [pltpu.async_copy(
        idf.at[idxid_v.at[pl.ds(t * 128, 128)]],
        idrow_v.at[pl.ds(t * 128, 128)], sem_id)
        for t in range(PB * 8 // 128)]

    def cnt_body(k, carry):
        i0 = k * 16
        cnt = jnp.zeros((16,), jnp.float32)
        for j in range(L):
            cnt = cnt + jnp.where(tok_v[j, pl.ds(i0, 16)] != 0, 1.0, 0.0)
        rec_v[pl.ds(i0, 16)] = 1.0 / jnp.maximum(cnt, jnp.float32(1.0))
        nz_v[pl.ds(i0, 16)] = jnp.float32(L) - cnt
        return carry

    lax.fori_loop(0, PB // 16, cnt_body, 0)

    def fire_group(g):
        buf = g % 2
        return [pltpu.async_copy(
            name2d.at[tok_v.at[j, pl.ds(g * CI, CI)]],
            rows[buf].at[pl.ds(j * CI, CI)], sem_n[buf])
            for j in range(L)]

    pend = fire_group(0)
    row0 = row0_v[0, :]
    for cp in id_cp:
        cp.wait()
    mrow = jnp.where(lane_lo, lane, lane + 16)
    nrow = lane + 8

    for g in range(G):
        buf = g % 2
        nxt = fire_group(g + 1) if g + 1 < G else None
        for cp in pend:
            cp.wait()
        pend = nxt

        def item_body(li, carry, g=g, buf=buf):
            gi = g * CI + li
            acc = rows[buf][li, :]
            for j in range(1, L):
                acc = acc + rows[buf][j * CI + li, :]
            gsp = jnp.full((16,), gi, dtype=jnp.int32)
            nz = plsc.load_gather(nz_v, [gsp])
            rec = plsc.load_gather(rec_v, [gsp])
            name = (acc - nz * row0) * rec
            idv16 = idrow_v[pl.ds(gi * 8, 16)]
            gsplat = plsc.load_gather(gid_v, [gsp])
            gicsv = plsc.load_gather(gicstab_v, [col8 * VG + gsplat])
            merged = jnp.where(lane_lo, idv16, gicsv)
            liv = jnp.full((16,), li, dtype=jnp.int32) + g * CI
            plsc.store_scatter(out_v, [mrow, liv], merged)
            plsc.store_scatter(out_v, [nrow, liv], name)
            return carry

        lax.fori_loop(0, CI, item_body, 0)

    pltpu.sync_copy(out_v, out.at[:, pl.ds(base, PB)])


@jax.jit
def _gather(idsf, tok2d, gicsids, idf, name2d, gicsf):
    mesh = plsc.VectorSubcoreMesh(**_MESH)
    return pl.kernel(
        _gather_body,
        out_type=jax.ShapeDtypeStruct((32, B), jnp.float32),
        mesh=mesh,
        scratch_types=[
            pltpu.VMEM((L, PB), jnp.int32),
            pltpu.VMEM((PB,), jnp.int32),
            pltpu.VMEM((PB,), jnp.int32),
            pltpu.VMEM((8 * VG,), jnp.float32),
            pltpu.VMEM((PB * 8,), jnp.int32),
            pltpu.VMEM((PB * 8 + 16,), jnp.float32),
            pltpu.VMEM((CI * L, 16), jnp.float32),
            pltpu.VMEM((CI * L, 16), jnp.float32),
            pltpu.VMEM((32, PB), jnp.float32),
            pltpu.VMEM((1, 16), jnp.float32),
            pltpu.VMEM((PB,), jnp.float32),
            pltpu.VMEM((PB,), jnp.float32),
            pltpu.SemaphoreType.DMA,
            pltpu.SemaphoreType.DMA,
            pltpu.SemaphoreType.DMA,
            pltpu.SemaphoreType.DMA,
        ],
        compiler_params=pltpu.CompilerParams(
            needs_layout_passes=False, use_tc_tiling_on_sc=False),
    )(idsf, tok2d, gicsids, idf, name2d, gicsf)


def kernel(item_id, item_name_tokens, item_gics, emb_id, emb_name, emb_gics):
    idsf = item_id.astype(jnp.int32)
    gicsids = item_gics.astype(jnp.int32)
    idtail = jnp.pad(emb_id[IDALIGN:], ((0, 40 - IDTAIL), (0, 0))).T.reshape(-1)
    nametail = emb_name[NMALIGN:].reshape(-1)
    gicsf = emb_gics.T.reshape(-1)
    tokf, idf, namef = _detile(
        item_name_tokens.astype(jnp.int32).T, emb_id.T, emb_name.T,
        idtail, nametail)
    outT = _gather(idsf, tokf.reshape(L, B), gicsids, idf,
                   namef.reshape(VT, 16), gicsf)
    return outT.T


# speedup vs baseline: 19.8320x; 1.0360x over previous
import jax
import jax.numpy as jnp
from jax import lax
from jax.experimental import pallas as pl
from jax.experimental.pallas import tpu as pltpu
from jax.experimental.pallas import tpu_sc as plsc

B = 16384
L = 20
VID = 100001
VIDP = 100008
VT = 10000
VG = 1001
VGP = 1008
NC, NS = 2, 16
NW = NC * NS
PB = B // NW

IDC = 3200
IDALIGN = 99968
IDTAIL = VID - IDALIGN
NMC = 384
NM_FULL = VT // NMC
NMALIGN = NM_FULL * NMC
NMTAIL = VT - NMALIGN

G = 4
CI = PB // G

_MESH = dict(core_axis_name="c", subcore_axis_name="s",
             num_cores=NC, num_subcores=NS)


def _detile_body(tokT, idT, nameT, idtail, nametail,
                 tokf, idf, namef,
                 tok_v, tokout_v, id_v, idout_v, nmin_v, nmout_v,
                 tail_v, ntail_v, sem_in, sem_out):
    wid = lax.axis_index("s") * NC + lax.axis_index("c")
    lane = lax.iota(jnp.int32, 16)

    pltpu.sync_copy(tokT.at[:, pl.ds(wid * PB, PB)], tok_v)

    def tok_body(c, carry):
        for j in range(L):
            tokout_v[pl.ds(j * PB + c * 16, 16)] = tok_v[j, pl.ds(c * 16, 16)]
        return carry

    lax.fori_loop(0, PB // 16, tok_body, 0)
    tok_cp = [pltpu.async_copy(tokout_v.at[pl.ds(j * PB, PB)],
                               tokf.at[pl.ds(j * B + wid * PB, PB)], sem_out)
              for j in range(L)]

    def _id_chunk(off, ncols):
        pltpu.sync_copy(idT.at[:, pl.ds(off, ncols)],
                        id_v.at[:, pl.ds(0, ncols)])
        d8 = lane & 7
        c2 = lane >> 3

        def id_body(c, carry):
            cc = c * 2
            v = plsc.load_gather(id_v, [d8, jnp.full((16,), cc, jnp.int32) + c2])
            idout_v[pl.ds(cc * 8, 16)] = v
            return carry

        lax.fori_loop(0, ncols // 2, id_body, 0)
        pltpu.sync_copy(idout_v.at[pl.ds(0, ncols * 8)],
                        idf.at[pl.ds(off * 8, ncols * 8)])

    @pl.when(wid < 31)
    def _id_full():
        _id_chunk(wid * IDC, IDC)

    @pl.when(wid == 31)
    def _id_last():
        _id_chunk(31 * IDC, IDALIGN - 31 * IDC)
        pltpu.sync_copy(idtail, tail_v)
        pltpu.sync_copy(tail_v, idf.at[pl.ds(IDALIGN * 8, IDTAIL * 8)])

    @pl.when(wid < NM_FULL)
    def _name_full():
        off_cols = wid * NMC
        pltpu.sync_copy(nameT.at[:, pl.ds(off_cols, NMC)], nmin_v)

        def row_body(t, carry):
            v = plsc.load_gather(nmin_v, [lane, jnp.full((16,), 0, jnp.int32) + t])
            nmout_v[pl.ds(t * 16, 16)] = v
            return carry

        lax.fori_loop(0, NMC, row_body, 0)
        pltpu.sync_copy(nmout_v, namef.at[pl.ds(off_cols * 16, NMC * 16)])

    @pl.when(wid == NM_FULL)
    def _name_last():
        pltpu.sync_copy(nametail, ntail_v)
        pltpu.sync_copy(ntail_v, namef.at[pl.ds(NMALIGN * 16, NMTAIL * 16)])

    for cp in tok_cp:
        cp.wait()


@jax.jit
def _detile(tokT, idT, nameT, idtail, nametail):
    mesh = plsc.VectorSubcoreMesh(**_MESH)
    return pl.kernel(
        _detile_body,
        out_type=(
            jax.ShapeDtypeStruct((B * L,), jnp.int32),
            jax.ShapeDtypeStruct((VID * 8,), jnp.float32),
            jax.ShapeDtypeStruct((VT * 16,), jnp.float32),
        ),
        mesh=mesh,
        scratch_types=[
            pltpu.VMEM((L, PB), jnp.int32),
            pltpu.VMEM((L * PB,), jnp.int32),
            pltpu.VMEM((8, IDC), jnp.float32),
            pltpu.VMEM((8 * IDC,), jnp.float32),
            pltpu.VMEM((16, NMC), jnp.float32),
            pltpu.VMEM((NMC * 16,), jnp.float32),
            pltpu.VMEM((IDTAIL * 8,), jnp.float32),
            pltpu.VMEM((NMTAIL * 16,), jnp.float32),
            pltpu.SemaphoreType.DMA,
            pltpu.SemaphoreType.DMA,
        ],
        compiler_params=pltpu.CompilerParams(
            needs_layout_passes=False, use_tc_tiling_on_sc=True),
    )(tokT, idT, nameT, idtail, nametail)


def _gather_body(idsf, tok2d, gicsids, id2d, name2d, gicsf, out,
                 tok_v, idid_v, gid_v, gicstab_v, idrow_v,
                 rows0, rows1, out_v, row0_v, rec_v, nz_v,
                 sem_id, sem_n0, sem_n1, sem_out):
    wid = lax.axis_index("s") * NC + lax.axis_index("c")
    base = wid * PB
    rows = (rows0, rows1)
    sem_n = (sem_n0, sem_n1)
    lane = lax.iota(jnp.int32, 16)
    lane_lo = lane < 8
    col8 = lane & 7

    pltpu.sync_copy(tok2d.at[:, pl.ds(base, PB)], tok_v)
    pltpu.sync_copy(idsf.at[pl.ds(base, PB)], idid_v)
    pltpu.sync_copy(gicsids.at[pl.ds(base, PB)], gid_v)
    pltpu.sync_copy(gicsf, gicstab_v)
    pltpu.sync_copy(name2d.at[pl.ds(0, 1)], row0_v)

    id_cp = [pltpu.async_copy(
        id2d.at[idid_v.at[pl.ds(t * 128, 128)]],
        idrow_v.at[pl.ds(t * 128, 128)], sem_id)
        for t in range(PB // 128)]

    def cnt_body(k, carry):
        i0 = k * 16
        cnt = jnp.zeros((16,), jnp.float32)
        for j in range(L):
            cnt = cnt + jnp.where(tok_v[j, pl.ds(i0, 16)] != 0, 1.0, 0.0)
        rec_v[pl.ds(i0, 16)] = 1.0 / jnp.maximum(cnt, jnp.float32(1.0))
        nz_v[pl.ds(i0, 16)] = jnp.float32(L) - cnt
        return carry

    lax.fori_loop(0, PB // 16, cnt_body, 0)

    def fire_group(g):
        buf = g % 2
        return [pltpu.async_copy(
            name2d.at[tok_v.at[j, pl.ds(g * CI, CI)]],
            rows[buf].at[pl.ds(j * CI, CI)], sem_n[buf])
            for j in range(L)]

    pend = fire_group(0)
    row0 = row0_v[0, :]
    for cp in id_cp:
        cp.wait()
    mrow = jnp.where(lane_lo, lane, lane + 16)
    nrow = lane + 8

    for g in range(G):
        buf = g % 2
        nxt = fire_group(g + 1) if g + 1 < G else None
        for cp in pend:
            cp.wait()
        pend = nxt

        def item_body(li, carry, g=g, buf=buf):
            gi = g * CI + li
            r = [rows[buf][j * CI + li, :] for j in range(L)]
            while len(r) > 1:
                r = [a + b for a, b in zip(r[::2], r[1::2])] + \
                    ([r[-1]] if len(r) % 2 else [])
            acc = r[0]
            gsp = jnp.full((16,), gi, dtype=jnp.int32)
            nz = plsc.load_gather(nz_v, [gsp])
            rec = plsc.load_gather(rec_v, [gsp])
            name = (acc - nz * row0) * rec
            idv16 = plsc.load_gather(idrow_v, [gsp, col8])
            gsplat = plsc.load_gather(gid_v, [gsp])
            gicsv = plsc.load_gather(gicstab_v, [col8 * VG + gsplat])
            merged = jnp.where(lane_lo, idv16, gicsv)
            liv = jnp.full((16,), gi, dtype=jnp.int32)
            plsc.store_scatter(out_v, [mrow, liv], merged)
            plsc.store_scatter(out_v, [nrow, liv], name)
            return carry

        lax.fori_loop(0, CI, item_body, 0)

    pltpu.sync_copy(out_v, out.at[:, pl.ds(base, PB)])


@jax.jit
def _gather(idsf, tok2d, gicsids, id2d, name2d, gicsf):
    mesh = plsc.VectorSubcoreMesh(**_MESH)
    return pl.kernel(
        _gather_body,
        out_type=jax.ShapeDtypeStruct((32, B), jnp.float32),
        mesh=mesh,
        scratch_types=[
            pltpu.VMEM((L, PB), jnp.int32),
            pltpu.VMEM((PB,), jnp.int32),
            pltpu.VMEM((PB,), jnp.int32),
            pltpu.VMEM((8 * VG,), jnp.float32),
            pltpu.VMEM((PB, 8), jnp.float32),
            pltpu.VMEM((CI * L, 16), jnp.float32),
            pltpu.VMEM((CI * L, 16), jnp.float32),
            pltpu.VMEM((32, PB), jnp.float32),
            pltpu.VMEM((1, 16), jnp.float32),
            pltpu.VMEM((PB,), jnp.float32),
            pltpu.VMEM((PB,), jnp.float32),
            pltpu.SemaphoreType.DMA,
            pltpu.SemaphoreType.DMA,
            pltpu.SemaphoreType.DMA,
            pltpu.SemaphoreType.DMA,
        ],
        compiler_params=pltpu.CompilerParams(
            needs_layout_passes=False, use_tc_tiling_on_sc=False),
    )(idsf, tok2d, gicsids, id2d, name2d, gicsf)


def kernel(item_id, item_name_tokens, item_gics, emb_id, emb_name, emb_gics):
    idsf = item_id.astype(jnp.int32)
    gicsids = item_gics.astype(jnp.int32)
    idtail = emb_id[IDALIGN:].reshape(-1)
    nametail = emb_name[NMALIGN:].reshape(-1)
    gicsf = emb_gics.T.reshape(-1)
    tokf, idf, namef = _detile(
        item_name_tokens.astype(jnp.int32).T, emb_id.T, emb_name.T,
        idtail, nametail)
    outT = _gather(idsf, tokf.reshape(L, B), gicsids, idf.reshape(VID, 8),
                   namef.reshape(VT, 16), gicsf)
    return outT.T
